# Initial kernel scaffold; baseline (speedup 1.0000x reference)
import jax
import jax.numpy as jnp
from jax.experimental import pallas as pl


def kernel(x, edge_index, edge_attr, We, be, Wn, bn):
    raise NotImplementedError("write your pallas kernel here")


# speedup vs baseline: 2.9043x; 2.9043x over previous
import functools

import jax
import jax.numpy as jnp
from jax import lax
from jax.experimental import pallas as pl
from jax.experimental.pallas import tpu as pltpu
from jax.experimental.pallas import tpu_sc as plsc

M_LAYERS = 10
D = 128
N_NODES = 10000
N_EDGES = 320000

NC = 2
NS = 16
NW = NC * NS
EPW = N_EDGES // NW
CHUNK = 96
NFULL = EPW // CHUNK
TAIL = EPW - NFULL * CHUNK
N_PAD = 10240
ROWS_PER_SUB = N_PAD // NS

NODE_BLK = 2000
EDGE_BLK = 2000


def _proj_body(x_ref, ws_ref, wd_ref, be_ref, ps_ref, pd_ref):
    x = x_ref[...]
    ps_ref[...] = jnp.dot(x, ws_ref[...], preferred_element_type=jnp.float32)
    pd_ref[...] = (jnp.dot(x, wd_ref[...], preferred_element_type=jnp.float32)
                   + be_ref[...])


def _proj(x, ws, wd, be):
    return pl.pallas_call(
        _proj_body,
        grid=(N_NODES // NODE_BLK,),
        in_specs=[pl.BlockSpec((NODE_BLK, D), lambda i: (i, 0)),
                  pl.BlockSpec((D, D), lambda i: (0, 0)),
                  pl.BlockSpec((D, D), lambda i: (0, 0)),
                  pl.BlockSpec((1, D), lambda i: (0, 0))],
        out_specs=[pl.BlockSpec((NODE_BLK, D), lambda i: (i, 0)),
                   pl.BlockSpec((NODE_BLK, D), lambda i: (i, 0))],
        out_shape=[jax.ShapeDtypeStruct((N_NODES, D), jnp.float32),
                   jax.ShapeDtypeStruct((N_NODES, D), jnp.float32)],
    )(x, ws, wd, be.reshape(1, D))


def _edge0_body(ea_ref, w_ref, pe_ref):
    pe_ref[...] = jnp.dot(ea_ref[...], w_ref[...],
                          preferred_element_type=jnp.float32)


def _edge0(ea, w):
    return pl.pallas_call(
        _edge0_body,
        grid=(N_EDGES // EDGE_BLK,),
        in_specs=[pl.BlockSpec((EDGE_BLK, D), lambda i: (i, 0)),
                  pl.BlockSpec((D, D), lambda i: (0, 0))],
        out_specs=pl.BlockSpec((EDGE_BLK, D), lambda i: (i, 0)),
        out_shape=jax.ShapeDtypeStruct((N_EDGES, D), jnp.float32),
    )(ea, w)


def _edge_body(ea_ref, eo_ref, w_ref, ea_out_ref, pe_ref):
    ea = ea_ref[...] + eo_ref[...]
    ea_out_ref[...] = ea
    pe_ref[...] = jnp.dot(ea, w_ref[...], preferred_element_type=jnp.float32)


def _edge(ea, eo, w):
    return pl.pallas_call(
        _edge_body,
        grid=(N_EDGES // EDGE_BLK,),
        in_specs=[pl.BlockSpec((EDGE_BLK, D), lambda i: (i, 0)),
                  pl.BlockSpec((EDGE_BLK, D), lambda i: (i, 0)),
                  pl.BlockSpec((D, D), lambda i: (0, 0))],
        out_specs=[pl.BlockSpec((EDGE_BLK, D), lambda i: (i, 0)),
                   pl.BlockSpec((EDGE_BLK, D), lambda i: (i, 0))],
        out_shape=[jax.ShapeDtypeStruct((N_EDGES, D), jnp.float32),
                   jax.ShapeDtypeStruct((N_EDGES, D), jnp.float32)],
    )(ea, eo, w)


def _resid_body(ea_ref, eo_ref, out_ref):
    out_ref[...] = ea_ref[...] + eo_ref[...]


def _resid(ea, eo):
    return pl.pallas_call(
        _resid_body,
        grid=(N_EDGES // EDGE_BLK,),
        in_specs=[pl.BlockSpec((EDGE_BLK, D), lambda i: (i, 0)),
                  pl.BlockSpec((EDGE_BLK, D), lambda i: (i, 0))],
        out_specs=pl.BlockSpec((EDGE_BLK, D), lambda i: (i, 0)),
        out_shape=jax.ShapeDtypeStruct((N_EDGES, D), jnp.float32),
    )(ea, eo)


def _node_body(x_ref, a0_ref, a1_ref, wx_ref, wa_ref, bn_ref, out_ref):
    x = x_ref[...]
    agg = a0_ref[...] + a1_ref[...]
    h = (jnp.dot(x, wx_ref[...], preferred_element_type=jnp.float32)
         + jnp.dot(agg, wa_ref[...], preferred_element_type=jnp.float32)
         + bn_ref[...])
    out_ref[...] = jnp.maximum(h, 0.0) + x


def _node(x, a0, a1, wx, wa, bn):
    return pl.pallas_call(
        _node_body,
        grid=(N_NODES // NODE_BLK,),
        in_specs=[pl.BlockSpec((NODE_BLK, D), lambda i: (i, 0)),
                  pl.BlockSpec((NODE_BLK, D), lambda i: (i, 0)),
                  pl.BlockSpec((NODE_BLK, D), lambda i: (i, 0)),
                  pl.BlockSpec((D, D), lambda i: (0, 0)),
                  pl.BlockSpec((D, D), lambda i: (0, 0)),
                  pl.BlockSpec((1, D), lambda i: (0, 0))],
        out_specs=pl.BlockSpec((NODE_BLK, D), lambda i: (i, 0)),
        out_shape=jax.ShapeDtypeStruct((N_NODES, D), jnp.float32),
    )(x, a0, a1, wx, wa, bn.reshape(1, D))


_MESH = plsc.VectorSubcoreMesh(core_axis_name="c", subcore_axis_name="s")


@functools.partial(
    pl.kernel,
    out_type=(jax.ShapeDtypeStruct((N_EDGES, D), jnp.float32),
              jax.ShapeDtypeStruct((NC, N_PAD, D), jnp.float32)),
    mesh=_MESH,
    scratch_types=[
        pltpu.VMEM((CHUNK,), jnp.int32),
        pltpu.VMEM((CHUNK,), jnp.int32),
        pltpu.VMEM((TAIL,), jnp.int32),
        pltpu.VMEM((TAIL,), jnp.int32),
        pltpu.VMEM((CHUNK, D), jnp.float32),
        pltpu.VMEM((CHUNK, D), jnp.float32),
        pltpu.VMEM((CHUNK, D), jnp.float32),
        pltpu.VMEM_SHARED((N_PAD, D), jnp.float32),
        pltpu.SemaphoreType.DMA,
        pltpu.SemaphoreType.DMA,
        pltpu.SemaphoreType.DMA,
    ],
)
def _sc_edge(ps_hbm, pd_hbm, pe_hbm, src_hbm, dst_hbm, zero_hbm,
             eo_hbm, agg_hbm,
             si_v, di_v, sit_v, dit_v, a_v, b_v, c_v,
             agg_sh, sem_a, sem_b, sem_c):
    cid = lax.axis_index("c")
    sid = lax.axis_index("s")
    wid = sid * NC + cid
    base = wid * EPW

    row0 = sid * ROWS_PER_SUB
    pltpu.sync_copy(zero_hbm.at[pl.ds(row0, ROWS_PER_SUB)],
                    agg_sh.at[pl.ds(row0, ROWS_PER_SUB)])
    plsc.subcore_barrier()

    def run_chunk(off, n, si, di, av, bv, cv):
        pltpu.sync_copy(src_hbm.at[pl.ds(off, n)], si)
        pltpu.sync_copy(dst_hbm.at[pl.ds(off, n)], di)
        cp_a = pltpu.async_copy(ps_hbm.at[si], av, sem_a)
        cp_b = pltpu.async_copy(pd_hbm.at[di], bv, sem_b)
        cp_c = pltpu.async_copy(pe_hbm.at[pl.ds(off, n)], cv, sem_c)
        cp_a.wait()
        cp_b.wait()
        cp_c.wait()

        @pl.loop(0, n)
        def _(r):
            for g in range(D // 16):
                s = pl.ds(g * 16, 16)
                cv[r, s] = jnp.maximum(av[r, s] + bv[r, s] + cv[r, s], 0.0)

        pltpu.sync_copy(cv, eo_hbm.at[pl.ds(off, n)])
        pltpu.sync_copy(cv, agg_sh.at[di], add=True)

    @pl.loop(0, NFULL * CHUNK, step=CHUNK)
    def _(o):
        run_chunk(base + o, CHUNK, si_v, di_v, a_v, b_v, c_v)
    run_chunk(base + NFULL * CHUNK, TAIL, sit_v, dit_v,
              a_v.at[pl.ds(0, TAIL)], b_v.at[pl.ds(0, TAIL)],
              c_v.at[pl.ds(0, TAIL)])

    plsc.subcore_barrier()
    pltpu.sync_copy(agg_sh.at[pl.ds(row0, ROWS_PER_SUB)],
                    agg_hbm.at[cid, pl.ds(row0, ROWS_PER_SUB)])


def kernel(x, edge_index, edge_attr, We, be, Wn, bn):
    src = edge_index[0].astype(jnp.int32)
    dst = edge_index[1].astype(jnp.int32)
    zero = jnp.zeros((N_PAD, D), jnp.float32)
    ea = edge_attr
    eo = None
    for i in range(M_LAYERS):
        ps, pd = _proj(x, We[i, :D], We[i, D:2 * D], be[i])
        if i == 0:
            pe = _edge0(ea, We[i, 2 * D:])
        else:
            ea, pe = _edge(ea, eo, We[i, 2 * D:])
        eo, agg = _sc_edge(ps, pd, pe, src, dst, zero)
        x = _node(x, agg[0, :N_NODES], agg[1, :N_NODES],
                  Wn[i, :D], Wn[i, D:], bn[i])
    ea = _resid(ea, eo)
    return x, ea


# speedup vs baseline: 4.1443x; 1.4269x over previous
import functools

import jax
import jax.numpy as jnp
from jax import lax
from jax.experimental import pallas as pl
from jax.experimental.pallas import tpu as pltpu
from jax.experimental.pallas import tpu_sc as plsc

M_LAYERS = 10
D = 128
N_NODES = 10000
N_EDGES = 320000

NC = 2
NS = 16
NW = NC * NS
EPW = N_EDGES // NW
CHUNK = 56
NFULL = EPW // CHUNK
TAIL = EPW - NFULL * CHUNK
N_PAD = 10240
ROWS_PER_SUB = N_PAD // NS

NODE_BLK = 2000
EDGE_BLK = 2000


def _proj_body(x_ref, ws_ref, wd_ref, be_ref, ps_ref, pd_ref):
    x = x_ref[...]
    ps_ref[...] = jnp.dot(x, ws_ref[...], preferred_element_type=jnp.float32)
    pd_ref[...] = (jnp.dot(x, wd_ref[...], preferred_element_type=jnp.float32)
                   + be_ref[...])


def _proj(x, ws, wd, be):
    return pl.pallas_call(
        _proj_body,
        grid=(N_NODES // NODE_BLK,),
        in_specs=[pl.BlockSpec((NODE_BLK, D), lambda i: (i, 0)),
                  pl.BlockSpec((D, D), lambda i: (0, 0)),
                  pl.BlockSpec((D, D), lambda i: (0, 0)),
                  pl.BlockSpec((1, D), lambda i: (0, 0))],
        out_specs=[pl.BlockSpec((NODE_BLK, D), lambda i: (i, 0)),
                   pl.BlockSpec((NODE_BLK, D), lambda i: (i, 0))],
        out_shape=[jax.ShapeDtypeStruct((N_NODES, D), jnp.float32),
                   jax.ShapeDtypeStruct((N_NODES, D), jnp.float32)],
    )(x, ws, wd, be.reshape(1, D))


def _edge0_body(ea_ref, w_ref, pe_ref):
    pe_ref[...] = jnp.dot(ea_ref[...], w_ref[...],
                          preferred_element_type=jnp.float32)


def _edge0(ea, w):
    return pl.pallas_call(
        _edge0_body,
        grid=(N_EDGES // EDGE_BLK,),
        in_specs=[pl.BlockSpec((EDGE_BLK, D), lambda i: (i, 0)),
                  pl.BlockSpec((D, D), lambda i: (0, 0))],
        out_specs=pl.BlockSpec((EDGE_BLK, D), lambda i: (i, 0)),
        out_shape=jax.ShapeDtypeStruct((N_EDGES, D), jnp.float32),
    )(ea, w)


def _edge_body(ea_ref, eo_ref, w_ref, ea_out_ref, pe_ref):
    ea = ea_ref[...] + eo_ref[...]
    ea_out_ref[...] = ea
    pe_ref[...] = jnp.dot(ea, w_ref[...], preferred_element_type=jnp.float32)


def _edge(ea, eo, w):
    return pl.pallas_call(
        _edge_body,
        grid=(N_EDGES // EDGE_BLK,),
        in_specs=[pl.BlockSpec((EDGE_BLK, D), lambda i: (i, 0)),
                  pl.BlockSpec((EDGE_BLK, D), lambda i: (i, 0)),
                  pl.BlockSpec((D, D), lambda i: (0, 0))],
        out_specs=[pl.BlockSpec((EDGE_BLK, D), lambda i: (i, 0)),
                   pl.BlockSpec((EDGE_BLK, D), lambda i: (i, 0))],
        out_shape=[jax.ShapeDtypeStruct((N_EDGES, D), jnp.float32),
                   jax.ShapeDtypeStruct((N_EDGES, D), jnp.float32)],
    )(ea, eo, w)


def _resid_body(ea_ref, eo_ref, out_ref):
    out_ref[...] = ea_ref[...] + eo_ref[...]


def _resid(ea, eo):
    return pl.pallas_call(
        _resid_body,
        grid=(N_EDGES // EDGE_BLK,),
        in_specs=[pl.BlockSpec((EDGE_BLK, D), lambda i: (i, 0)),
                  pl.BlockSpec((EDGE_BLK, D), lambda i: (i, 0))],
        out_specs=pl.BlockSpec((EDGE_BLK, D), lambda i: (i, 0)),
        out_shape=jax.ShapeDtypeStruct((N_EDGES, D), jnp.float32),
    )(ea, eo)


def _node_body(x_ref, a0_ref, a1_ref, wx_ref, wa_ref, bn_ref, out_ref):
    x = x_ref[...]
    agg = a0_ref[...] + a1_ref[...]
    h = (jnp.dot(x, wx_ref[...], preferred_element_type=jnp.float32)
         + jnp.dot(agg, wa_ref[...], preferred_element_type=jnp.float32)
         + bn_ref[...])
    out_ref[...] = jnp.maximum(h, 0.0) + x


def _node(x, a0, a1, wx, wa, bn):
    return pl.pallas_call(
        _node_body,
        grid=(N_NODES // NODE_BLK,),
        in_specs=[pl.BlockSpec((NODE_BLK, D), lambda i: (i, 0)),
                  pl.BlockSpec((NODE_BLK, D), lambda i: (i, 0)),
                  pl.BlockSpec((NODE_BLK, D), lambda i: (i, 0)),
                  pl.BlockSpec((D, D), lambda i: (0, 0)),
                  pl.BlockSpec((D, D), lambda i: (0, 0)),
                  pl.BlockSpec((1, D), lambda i: (0, 0))],
        out_specs=pl.BlockSpec((NODE_BLK, D), lambda i: (i, 0)),
        out_shape=jax.ShapeDtypeStruct((N_NODES, D), jnp.float32),
    )(x, a0, a1, wx, wa, bn.reshape(1, D))


_MESH = plsc.VectorSubcoreMesh(core_axis_name="c", subcore_axis_name="s")


@functools.partial(
    pl.kernel,
    out_type=(jax.ShapeDtypeStruct((N_EDGES, D), jnp.float32),
              jax.ShapeDtypeStruct((NC, N_PAD, D), jnp.float32)),
    mesh=_MESH,
    scratch_types=[
        pltpu.VMEM((CHUNK,), jnp.int32),
        pltpu.VMEM((CHUNK,), jnp.int32),
        pltpu.VMEM((CHUNK,), jnp.int32),
        pltpu.VMEM((CHUNK,), jnp.int32),
        pltpu.VMEM((TAIL,), jnp.int32),
        pltpu.VMEM((TAIL,), jnp.int32),
        pltpu.VMEM((CHUNK, D), jnp.float32),
        pltpu.VMEM((CHUNK, D), jnp.float32),
        pltpu.VMEM((CHUNK, D), jnp.float32),
        pltpu.VMEM((CHUNK, D), jnp.float32),
        pltpu.VMEM((CHUNK, D), jnp.float32),
        pltpu.VMEM((CHUNK, D), jnp.float32),
        pltpu.VMEM_SHARED((N_PAD, D), jnp.float32),
        pltpu.SemaphoreType.DMA,
        pltpu.SemaphoreType.DMA,
        pltpu.SemaphoreType.DMA,
        pltpu.SemaphoreType.DMA,
        pltpu.SemaphoreType.DMA,
        pltpu.SemaphoreType.DMA,
    ],
)
def _sc_edge(ps_hbm, pd_hbm, pe_hbm, src_hbm, dst_hbm, zero_hbm,
             eo_hbm, agg_hbm,
             si0, si1, di0, di1, sit, dit, a0, b0, c0, a1, b1, c1,
             agg_sh, gsem0, gsem1, osem0, osem1, isem0, isem1):
    cid = lax.axis_index("c")
    sid = lax.axis_index("s")
    wid = sid * NC + cid
    base = wid * EPW
    L = NFULL * CHUNK

    sets = ((si0, di0, a0, b0, c0, gsem0, osem0, isem0),
            (si1, di1, a1, b1, c1, gsem1, osem1, isem1))

    row0 = sid * ROWS_PER_SUB
    pltpu.sync_copy(zero_hbm.at[pl.ds(row0, ROWS_PER_SUB)],
                    agg_sh.at[pl.ds(row0, ROWS_PER_SUB)])
    plsc.subcore_barrier()

    def issue_idx(lo, s):
        sv, dv = sets[s][0], sets[s][1]
        isem = sets[s][7]
        pltpu.async_copy(src_hbm.at[pl.ds(base + lo, CHUNK)], sv, isem)
        pltpu.async_copy(dst_hbm.at[pl.ds(base + lo, CHUNK)], dv, isem)

    def wait_idx(s):
        sv, dv = sets[s][0], sets[s][1]
        isem = sets[s][7]
        pltpu.make_async_copy(src_hbm.at[pl.ds(base, CHUNK)], sv, isem).wait()
        pltpu.make_async_copy(dst_hbm.at[pl.ds(base, CHUNK)], dv, isem).wait()

    def issue_gathers(lo, s, drain_store):
        sv, dv, av, bv, cv, gsem, osem, _ = sets[s]

        @pl.when(drain_store)
        def _():
            pltpu.make_async_copy(cv, eo_hbm.at[pl.ds(base, CHUNK)],
                                  osem).wait()

        pltpu.async_copy(ps_hbm.at[sv], av, gsem)
        pltpu.async_copy(pd_hbm.at[dv], bv, gsem)
        pltpu.async_copy(pe_hbm.at[pl.ds(base + lo, CHUNK)], cv, gsem)

    def wait_gathers(lo, s):
        sv, dv, av, bv, cv, gsem, _, _ = sets[s]
        pltpu.make_async_copy(ps_hbm.at[sv], av, gsem).wait()
        pltpu.make_async_copy(pd_hbm.at[dv], bv, gsem).wait()
        pltpu.make_async_copy(pe_hbm.at[pl.ds(base + lo, CHUNK)], cv,
                              gsem).wait()

    def process(lo, s):
        sv, dv, av, bv, cv, _, osem, _ = sets[s]

        @pl.loop(0, CHUNK)
        def _(r):
            for g in range(D // 16):
                sl = pl.ds(g * 16, 16)
                cv[r, sl] = jnp.maximum(av[r, sl] + bv[r, sl] + cv[r, sl], 0.0)

        pltpu.sync_copy(cv, agg_sh.at[dv], add=True)
        pltpu.async_copy(cv, eo_hbm.at[pl.ds(base + lo, CHUNK)], osem)

    issue_idx(0, 0)
    wait_idx(0)
    issue_gathers(0, 0, False)
    issue_idx(CHUNK, 1)

    @pl.loop(0, L, step=2 * CHUNK)
    def _(lo):
        wait_idx(1)
        issue_gathers(lo + CHUNK, 1, lo > 0)
        wait_gathers(lo, 0)
        process(lo, 0)
        more = lo + 2 * CHUNK < L

        @pl.when(more)
        def _():
            issue_idx(lo + 2 * CHUNK, 0)

        @pl.when(more)
        def _():
            wait_idx(0)
            issue_gathers(lo + 2 * CHUNK, 0, True)
        wait_gathers(lo + CHUNK, 1)
        process(lo + CHUNK, 1)

        @pl.when(lo + 3 * CHUNK < L)
        def _():
            issue_idx(lo + 3 * CHUNK, 1)

    pltpu.make_async_copy(c0, eo_hbm.at[pl.ds(base, CHUNK)], osem0).wait()
    pltpu.make_async_copy(c1, eo_hbm.at[pl.ds(base, CHUNK)], osem1).wait()

    to = base + L
    pltpu.sync_copy(src_hbm.at[pl.ds(to, TAIL)], sit)
    pltpu.sync_copy(dst_hbm.at[pl.ds(to, TAIL)], dit)
    at, bt, ct = (a0.at[pl.ds(0, TAIL)], b0.at[pl.ds(0, TAIL)],
                  c0.at[pl.ds(0, TAIL)])
    cp_a = pltpu.async_copy(ps_hbm.at[sit], at, gsem0)
    cp_b = pltpu.async_copy(pd_hbm.at[dit], bt, gsem1)
    cp_c = pltpu.async_copy(pe_hbm.at[pl.ds(to, TAIL)], ct, osem0)
    cp_a.wait()
    cp_b.wait()
    cp_c.wait()

    @pl.loop(0, TAIL)
    def _(r):
        for g in range(D // 16):
            sl = pl.ds(g * 16, 16)
            ct[r, sl] = jnp.maximum(at[r, sl] + bt[r, sl] + ct[r, sl], 0.0)

    pltpu.sync_copy(ct, eo_hbm.at[pl.ds(to, TAIL)])
    pltpu.sync_copy(ct, agg_sh.at[dit], add=True)

    plsc.subcore_barrier()
    pltpu.sync_copy(agg_sh.at[pl.ds(row0, ROWS_PER_SUB)],
                    agg_hbm.at[cid, pl.ds(row0, ROWS_PER_SUB)])


def kernel(x, edge_index, edge_attr, We, be, Wn, bn):
    src = edge_index[0].astype(jnp.int32)
    dst = edge_index[1].astype(jnp.int32)
    zero = jnp.zeros((N_PAD, D), jnp.float32)
    ea = edge_attr
    eo = None
    for i in range(M_LAYERS):
        ps, pd = _proj(x, We[i, :D], We[i, D:2 * D], be[i])
        if i == 0:
            pe = _edge0(ea, We[i, 2 * D:])
        else:
            ea, pe = _edge(ea, eo, We[i, 2 * D:])
        eo, agg = _sc_edge(ps, pd, pe, src, dst, zero)
        x = _node(x, agg[0, :N_NODES], agg[1, :N_NODES],
                  Wn[i, :D], Wn[i, D:], bn[i])
    ea = _resid(ea, eo)
    return x, ea
